# Initial kernel scaffold; baseline (speedup 1.0000x reference)
import jax
import jax.numpy as jnp
from jax.experimental import pallas as pl


def kernel(x, edge_index, edge_attr, W_in, b_in, W1, b1, W2, b2, W_mu, b_mu, W_lv, b_lv, g1, be1, g2, be2):
    raise NotImplementedError("write your pallas kernel here")


# speedup vs baseline: 6.5643x; 6.5643x over previous
import functools

import jax
import jax.numpy as jnp
from jax import lax
from jax.experimental import pallas as pl
from jax.experimental.pallas import tpu as pltpu
from jax.experimental.pallas import tpu_sc as plsc

_N = 10000
_D = 128
_NPAD = 10240
_ROWS = _NPAD // 128
_CH = 128
_NCHUNK = 79
_EPW = _CH * _NCHUNK
_NW = 32
_EPAD = _NW * _EPW
_SL = _NPAD // 16

_mesh = plsc.VectorSubcoreMesh(core_axis_name="c", subcore_axis_name="s")


def _deg_body(src_hbm, dst_hbm, ew_hbm, out_hbm, sidx, didx, ewv, zbuf, degs, degd):
    c = lax.axis_index("c")
    s = lax.axis_index("s")
    wid = c * 16 + s
    for i in range(_SL // 16):
        zbuf[pl.ds(i * 16, 16)] = jnp.zeros((16,), jnp.float32)
    pltpu.sync_copy(zbuf, degs.at[pl.ds(s * _SL, _SL)])
    pltpu.sync_copy(zbuf, degd.at[pl.ds(s * _SL, _SL)])
    plsc.subcore_barrier()

    def chunk(i, carry):
        off = wid * _EPW + i * _CH
        pltpu.sync_copy(src_hbm.at[pl.ds(off, _CH)], sidx)
        pltpu.sync_copy(dst_hbm.at[pl.ds(off, _CH)], didx)
        pltpu.sync_copy(ew_hbm.at[pl.ds(off, _CH)], ewv)
        pltpu.sync_copy(ewv, degs.at[sidx], add=True)
        pltpu.sync_copy(ewv, degd.at[didx], add=True)
        return carry

    lax.fori_loop(0, _NCHUNK, chunk, 0)
    plsc.subcore_barrier()
    pltpu.sync_copy(degs.at[pl.ds(s * _SL, _SL)], out_hbm.at[c, 0, pl.ds(s * _SL, _SL)])
    pltpu.sync_copy(degd.at[pl.ds(s * _SL, _SL)], out_hbm.at[c, 1, pl.ds(s * _SL, _SL)])


_deg_call = functools.partial(
    pl.kernel,
    mesh=_mesh,
    out_type=jax.ShapeDtypeStruct((2, 2, _NPAD), jnp.float32),
    scratch_types=[
        pltpu.VMEM((_CH,), jnp.int32),
        pltpu.VMEM((_CH,), jnp.int32),
        pltpu.VMEM((_CH,), jnp.float32),
        pltpu.VMEM((_SL,), jnp.float32),
        pltpu.VMEM_SHARED((_NPAD,), jnp.float32),
        pltpu.VMEM_SHARED((_NPAD,), jnp.float32),
    ],
)(_deg_body)


def _spmm_body(src_hbm, dst_hbm, ew_hbm, msg_hbm, out_hbm, sidx, didx, ewv, rows, acc, sem):
    c = lax.axis_index("c")
    s = lax.axis_index("s")
    wid = c * 16 + s

    def zrow(r, carry):
        for f in range(8):
            rows[r, pl.ds(f * 16, 16)] = jnp.zeros((16,), jnp.float32)
        return carry

    lax.fori_loop(0, _CH, zrow, 0)
    for k in range(_SL // _CH):
        pltpu.sync_copy(rows, acc.at[pl.ds(s * _SL + k * _CH, _CH)])
    plsc.subcore_barrier()

    def chunk(i, carry):
        off = wid * _EPW + i * _CH
        pltpu.sync_copy(src_hbm.at[pl.ds(off, _CH)], sidx)
        pltpu.sync_copy(dst_hbm.at[pl.ds(off, _CH)], didx)
        pltpu.sync_copy(ew_hbm.at[pl.ds(off, _CH)], ewv)
        pltpu.async_copy(msg_hbm.at[sidx], rows, sem).wait()

        def grp(g, carry2):
            ewg = ewv[pl.ds(g * 16, 16)]
            for i16 in range(16):
                e = g * 16 + i16
                w = ewg[i16]
                for f in range(8):
                    rows[e, pl.ds(f * 16, 16)] = rows[e, pl.ds(f * 16, 16)] * w
            return carry2

        lax.fori_loop(0, 8, grp, 0)
        pltpu.sync_copy(rows, acc.at[didx], add=True)
        return carry

    lax.fori_loop(0, _NCHUNK, chunk, 0)
    plsc.subcore_barrier()
    for k in range(_SL // _CH):
        pltpu.sync_copy(
            acc.at[pl.ds(s * _SL + k * _CH, _CH)],
            out_hbm.at[c, pl.ds(s * _SL + k * _CH, _CH)],
        )


_spmm_call = functools.partial(
    pl.kernel,
    mesh=_mesh,
    out_type=jax.ShapeDtypeStruct((2, _NPAD, _D), jnp.float32),
    scratch_types=[
        pltpu.VMEM((_CH,), jnp.int32),
        pltpu.VMEM((_CH,), jnp.int32),
        pltpu.VMEM((_CH,), jnp.float32),
        pltpu.VMEM((_CH, _D), jnp.float32),
        pltpu.VMEM_SHARED((_NPAD, _D), jnp.float32),
        pltpu.SemaphoreType.DMA,
    ],
)(_spmm_body)


def _rsqrt_body(degp_ref, rsd_ref):
    rsd_ref[...] = lax.rsqrt(degp_ref[0] + degp_ref[1] + 1e-6)


def _rsqrt_call(degp4):
    return pl.pallas_call(
        _rsqrt_body,
        out_shape=jax.ShapeDtypeStruct((2, _ROWS, 128), jnp.float32),
    )(degp4)


def _inproj_body(x_ref, win_ref, bin_ref, w1_ref, b1_ref, rs_ref, ms_ref):
    h = jnp.dot(x_ref[...], win_ref[...], preferred_element_type=jnp.float32) + bin_ref[...]
    m = jnp.dot(h, w1_ref[...], preferred_element_type=jnp.float32) + b1_ref[...]
    ms_ref[...] = m * rs_ref[...]


def _inproj_call(x, W_in, b_in, W1, b1, rsd_s):
    return pl.pallas_call(
        _inproj_body,
        out_shape=jax.ShapeDtypeStruct((_N, _D), jnp.float32),
    )(x, W_in, b_in, W1, b1, rsd_s)


def _mid_body(a0_ref, a1_ref, rd_ref, g_ref, be_ref, w_ref, b_ref, rs_ref, out_ref):
    cv = (a0_ref[...] + a1_ref[...]) * rd_ref[...]
    m = jnp.mean(cv, axis=0)
    v = jnp.mean((cv - m) ** 2, axis=0)
    h = jnp.maximum((cv - m) * lax.rsqrt(v + 1e-5) * g_ref[...] + be_ref[...], 0.0)
    out_ref[...] = (jnp.dot(h, w_ref[...], preferred_element_type=jnp.float32) + b_ref[...]) * rs_ref[...]


def _mid_call(a0, a1, rd, g, be, w, b, rs):
    return pl.pallas_call(
        _mid_body,
        out_shape=jax.ShapeDtypeStruct((_N, _D), jnp.float32),
    )(a0, a1, rd, g, be, w, b, rs)


def _fin_body(a0_ref, a1_ref, rd_ref, out_ref):
    out_ref[...] = (a0_ref[...] + a1_ref[...]) * rd_ref[...]


def _fin_call(a0, a1, rd):
    return pl.pallas_call(
        _fin_body,
        out_shape=jax.ShapeDtypeStruct((_N, _D), jnp.float32),
    )(a0, a1, rd)


def kernel(x, edge_index, edge_attr, W_in, b_in, W1, b1, W2, b2, W_mu, b_mu, W_lv, b_lv, g1, be1, g2, be2):
    src = edge_index[0]
    dst = edge_index[1]
    pad = _EPAD - src.shape[0]
    srcp = jnp.pad(src, (0, pad))
    dstp = jnp.pad(dst, (0, pad))
    ewp = jnp.pad(edge_attr, (0, pad))

    degp = _deg_call(srcp, dstp, ewp)
    rsd = _rsqrt_call(degp.reshape(2, 2, _ROWS, 128))
    rsd_s = rsd[0].reshape(_NPAD, 1)[:_N]
    rsd_d = rsd[1].reshape(_NPAD, 1)[:_N]

    ms1 = _inproj_call(x, W_in, b_in, W1, b1, rsd_s)
    acc1 = _spmm_call(srcp, dstp, ewp, ms1)
    ms2 = _mid_call(acc1[0, :_N], acc1[1, :_N], rsd_d, g1, be1, W2, b2, rsd_s)
    acc2 = _spmm_call(srcp, dstp, ewp, ms2)
    Wc = jnp.concatenate([W_mu, W_lv], axis=1)
    bc = jnp.concatenate([b_mu, b_lv])
    ms3 = _mid_call(acc2[0, :_N], acc2[1, :_N], rsd_d, g2, be2, Wc, bc, rsd_s)
    acc3 = _spmm_call(srcp, dstp, ewp, ms3)
    full = _fin_call(acc3[0, :_N], acc3[1, :_N], rsd_d)
    return (full[:, :64], full[:, 64:])
